# baseline (speedup 1.0000x reference)
import functools
import math

import jax
import jax.numpy as jnp
from jax import lax
from jax.experimental import pallas as pl
from jax.experimental.pallas import tpu as pltpu
from jax.experimental.pallas import tpu_sc as plsc

V = 100000
N = 1024
PAD = 0
SMOOTHING = 0.1
CONFIDENCE = 1.0 - SMOOTHING
SMOOTH_VAL = SMOOTHING / (V - 2)
C1 = (V - 2) * SMOOTH_VAL * math.log(SMOOTH_VAL) + CONFIDENCE * math.log(CONFIDENCE)


_CB = 2000
_NVB = V // (2 * _CB)


def _tc_stream_body(pa_ref, pb_ref, tgt_ref, s_ref, pt_ref, p0_ref,
                    accs_ref, accp_ref, acc0_ref):
    j = pl.program_id(0)

    @pl.when(j == 0)
    def _init():
        accs_ref[...] = jnp.zeros_like(accs_ref)
        accp_ref[...] = jnp.zeros_like(accp_ref)

    t = tgt_ref[...]
    acc_s = jnp.zeros((1, N), jnp.float32)
    acc_p = jnp.zeros((1, N), jnp.float32)
    for half, p_ref in ((0, pa_ref), (1, pb_ref)):
        blk = p_ref[...]
        vbase = (j + half * _NVB) * _CB
        viota = vbase + lax.broadcasted_iota(jnp.int32, (_CB, N), 0)
        acc_s += jnp.sum(blk, axis=0, keepdims=True)
        acc_p += jnp.sum(
            jnp.where(viota == t, blk, 0.0), axis=0, keepdims=True
        )
    accs_ref[...] += acc_s
    accp_ref[...] += acc_p

    @pl.when(j == 0)
    def _p0():
        acc0_ref[...] = pa_ref[0:1, :]

    @pl.when(j == _NVB - 1)
    def _emit():
        s_ref[...] = accs_ref[...]
        pt_ref[...] = accp_ref[...]
        p0_ref[...] = acc0_ref[...]


_tc_stream = pl.pallas_call(
    _tc_stream_body,
    grid=(_NVB,),
    in_specs=[
        pl.BlockSpec((_CB, N), lambda j: (j, 0)),
        pl.BlockSpec((_CB, N), lambda j: (j + _NVB, 0)),
        pl.BlockSpec((1, N), lambda j: (0, 0)),
    ],
    out_specs=[pl.BlockSpec((1, N), lambda j: (0, 0))] * 3,
    out_shape=[jax.ShapeDtypeStruct((1, N), jnp.float32)] * 3,
    scratch_shapes=[pltpu.VMEM((1, N), jnp.float32)] * 3,
    compiler_params=pltpu.CompilerParams(
        dimension_semantics=("arbitrary",),
    ),
)


_NS = 16
_RPT = N // _NS


def _sc_combine_body(s_hbm, pt_hbm, p0_hbm, tgt_hbm, part_hbm, fin_hbm,
                     s_v, pt_v, p0_v, tgt_v, acc_v, big_v, out_v):
    cid = lax.axis_index("c")
    sid = lax.axis_index("s")

    @pl.when(cid == 0)
    def _work():
        base = sid * _RPT
        pltpu.sync_copy(s_hbm.at[0, pl.ds(base, _RPT)], s_v)
        pltpu.sync_copy(pt_hbm.at[0, pl.ds(base, _RPT)], pt_v)
        pltpu.sync_copy(p0_hbm.at[0, pl.ds(base, _RPT)], p0_v)
        pltpu.sync_copy(tgt_hbm.at[pl.ds(base, _RPT)], tgt_v)
        acc = jnp.zeros((16,), jnp.float32)
        for k in range(_RPT // 16):
            sl = pl.ds(k * 16, 16)
            row_loss = (
                C1
                + SMOOTH_VAL * p0_v[sl]
                + (SMOOTH_VAL - CONFIDENCE) * pt_v[sl]
                - SMOOTH_VAL * s_v[sl]
            )
            acc = acc + jnp.where(tgt_v[sl] != PAD, row_loss, 0.0)
        acc_v[...] = acc
        pltpu.sync_copy(acc_v, part_hbm.at[sid])

    plsc.subcore_barrier()

    @pl.when((cid == 0) & (sid == 0))
    def _final():
        pltpu.sync_copy(part_hbm, big_v)
        tot = jnp.zeros((16,), jnp.float32)
        for i in range(_NS):
            tot = tot + big_v[i]
        lane = lax.iota(jnp.int32, 16)
        dn = lax.GatherDimensionNumbers(
            offset_dims=(), collapsed_slice_dims=(0,), start_index_map=(0,)
        )
        for st in (8, 4, 2, 1):
            tot = tot + lax.gather(
                tot, (lane ^ st).reshape(16, 1), dn, (1,),
                mode=lax.GatherScatterMode.PROMISE_IN_BOUNDS,
            )
        out_v[...] = tot
        pltpu.sync_copy(out_v, fin_hbm)


@functools.cache
def _sc_combine():
    return pl.kernel(
        _sc_combine_body,
        out_type=[
            jax.ShapeDtypeStruct((_NS, 16), jnp.float32),
            jax.ShapeDtypeStruct((16,), jnp.float32),
        ],
        mesh=plsc.VectorSubcoreMesh(core_axis_name="c", subcore_axis_name="s"),
        scratch_types=[
            pltpu.VMEM((_RPT,), jnp.float32),
            pltpu.VMEM((_RPT,), jnp.float32),
            pltpu.VMEM((_RPT,), jnp.float32),
            pltpu.VMEM((_RPT,), jnp.int32),
            pltpu.VMEM((16,), jnp.float32),
            pltpu.VMEM((_NS, 16), jnp.float32),
            pltpu.VMEM((16,), jnp.float32),
        ],
    )


def kernel(trg_tokens_probas, target_token_idxs):
    xt = trg_tokens_probas.T
    s, pt, p0 = _tc_stream(xt, xt, target_token_idxs.reshape(1, N))
    _, fin = _sc_combine()(s, pt, p0, target_token_idxs)
    return fin[0]


# speedup vs baseline: 1.0046x; 1.0046x over previous
import functools
import math

import jax
import jax.numpy as jnp
from jax import lax
from jax.experimental import pallas as pl
from jax.experimental.pallas import tpu as pltpu
from jax.experimental.pallas import tpu_sc as plsc

V = 100000
N = 1024
PAD = 0
SMOOTHING = 0.1
CONFIDENCE = 1.0 - SMOOTHING
SMOOTH_VAL = SMOOTHING / (V - 2)
C1 = (V - 2) * SMOOTH_VAL * math.log(SMOOTH_VAL) + CONFIDENCE * math.log(CONFIDENCE)


_CB = 4000
_NVB = V // _CB


def _tc_stream_body(p_ref, tgt_ref, spp_ref,
                    accs_ref, accp_ref, acc0_ref):
    j = pl.program_id(0)

    @pl.when(j == 0)
    def _init():
        accs_ref[...] = jnp.zeros_like(accs_ref)
        accp_ref[...] = jnp.zeros_like(accp_ref)

    blk = p_ref[...]
    viota = j * _CB + lax.broadcasted_iota(jnp.int32, (_CB, N), 0)
    accs_ref[...] += jnp.sum(blk, axis=0, keepdims=True)
    accp_ref[...] += jnp.sum(
        jnp.where(viota == tgt_ref[...], blk, 0.0), axis=0, keepdims=True
    )

    @pl.when(j == 0)
    def _p0():
        acc0_ref[...] = blk[0:1, :]

    @pl.when(j == _NVB - 1)
    def _emit():
        spp_ref[0:1, :] = accs_ref[...]
        spp_ref[1:2, :] = accp_ref[...]
        spp_ref[2:3, :] = acc0_ref[...]


_tc_stream = pl.pallas_call(
    _tc_stream_body,
    grid=(_NVB,),
    in_specs=[
        pl.BlockSpec((_CB, N), lambda j: (j, 0)),
        pl.BlockSpec((1, N), lambda j: (0, 0)),
    ],
    out_specs=pl.BlockSpec((8, N), lambda j: (0, 0)),
    out_shape=jax.ShapeDtypeStruct((8, N), jnp.float32),
    scratch_shapes=[pltpu.VMEM((1, N), jnp.float32)] * 3,
    compiler_params=pltpu.CompilerParams(
        dimension_semantics=("arbitrary",),
    ),
)


_NS = 16
_RPT = N // _NS


def _sc_combine_body(spp_hbm, tgt_hbm, fin_hbm, spp_v, tgt_v, out_v, sem):
    cid = lax.axis_index("c")
    sid = lax.axis_index("s")

    @pl.when((cid == 0) & (sid == 0))
    def _work():
        pltpu.sync_copy(spp_hbm, spp_v)
        pltpu.sync_copy(tgt_hbm, tgt_v)
        tot = jnp.zeros((16,), jnp.float32)
        for k in range(N // 16):
            sl = pl.ds(k * 16, 16)
            row_loss = (
                C1
                + SMOOTH_VAL * spp_v[2, sl]
                + (SMOOTH_VAL - CONFIDENCE) * spp_v[1, sl]
                - SMOOTH_VAL * spp_v[0, sl]
            )
            tot = tot + jnp.where(tgt_v[sl] != PAD, row_loss, 0.0)
        lane = lax.iota(jnp.int32, 16)
        dn = lax.GatherDimensionNumbers(
            offset_dims=(), collapsed_slice_dims=(0,), start_index_map=(0,)
        )
        for st in (8, 4, 2, 1):
            tot = tot + lax.gather(
                tot, (lane ^ st).reshape(16, 1), dn, (1,),
                mode=lax.GatherScatterMode.PROMISE_IN_BOUNDS,
            )
        out_v[...] = tot
        pltpu.sync_copy(out_v, fin_hbm)


@functools.cache
def _sc_combine():
    return pl.kernel(
        _sc_combine_body,
        out_type=jax.ShapeDtypeStruct((16,), jnp.float32),
        mesh=plsc.VectorSubcoreMesh(core_axis_name="c", subcore_axis_name="s"),
        scratch_types=[
            pltpu.VMEM((8, N), jnp.float32),
            pltpu.VMEM((N,), jnp.int32),
            pltpu.VMEM((16,), jnp.float32),
            pltpu.SemaphoreType.DMA,
        ],
    )


def kernel(trg_tokens_probas, target_token_idxs):
    spp = _tc_stream(trg_tokens_probas.T, target_token_idxs.reshape(1, N))
    fin = _sc_combine()(spp, target_token_idxs)
    return fin[0]
